# baseline (speedup 1.0000x reference)
import functools

import jax
import jax.numpy as jnp
from jax import lax
from jax.experimental import pallas as pl
from jax.experimental.pallas import tpu as pltpu
from jax.experimental.pallas import tpu_sc as plsc

N = 10000
E = 160000
NEI = 16
NODE_FDIM = 128
INPUT = 128
HIDDEN = 64

F32 = jnp.float32

NC = 2
NS = 16
NW = NC * NS

EPW = E // NW
GB = 8
CHE = 200
NCHUNK = EPW // CHE
GBD = 8
IBLK = GBD * NEI
CHD = CHE // GBD
NB = 5
AH = 4

_sc_mesh = plsc.VectorSubcoreMesh(core_axis_name="c", subcore_axis_name="s")


@functools.partial(
    pl.kernel,
    out_type=jax.ShapeDtypeStruct((E, 2 * HIDDEN), F32),
    mesh=_sc_mesh,
    scratch_types=[
        pltpu.VMEM((CHE * NEI,), jnp.int32),
        pltpu.VMEM((CHE, HIDDEN), F32),
        pltpu.VMEM((NB, IBLK, 2 * HIDDEN), F32),
        pltpu.VMEM((NB, GBD, 2 * HIDDEN), F32),
        pltpu.SemaphoreType.DMA,
        pltpu.SemaphoreType.DMA,
        pltpu.SemaphoreType.DMA,
        pltpu.SemaphoreType.DMA,
        pltpu.SemaphoreType.DMA,
        pltpu.SemaphoreType.DMA,
        pltpu.SemaphoreType.DMA,
        pltpu.SemaphoreType.DMA,
        pltpu.SemaphoreType.DMA,
        pltpu.SemaphoreType.DMA,
        pltpu.SemaphoreType.DMA,
        pltpu.SemaphoreType.DMA,
    ],
)
def _sc_depth(hcat_hbm, bgf_hbm, nr1_hbm, sumc_hbm,
              idx_v, nr1_v, rows_v, outc_v,
              sem0, sem1, sem2, sem3, sem4, semo0, semo1, semo2, semo3, semo4,
              semi, semn):
    wid = lax.axis_index("c") * NS + lax.axis_index("s")
    sems = (sem0, sem1, sem2, sem3, sem4)
    semos = (semo0, semo1, semo2, semo3, semo4)

    def _compute(b, buf, obuf):
        NCH = HIDDEN // 16
        EU = 2

        @pl.loop(0, GBD, step=EU)
        def _edge(e):
            rows = [e + u for u in range(EU)]
            pe = [[nr1_v[b * GBD + ee, pl.ds(c * 16, 16)] for c in range(NCH)]
                  for ee in rows]
            acc_s = [[jnp.zeros((16,), F32) for _ in range(NCH)]
                     for _ in range(EU)]
            acc_g = [[jnp.zeros((16,), F32) for _ in range(NCH)]
                     for _ in range(EU)]
            for k in range(NEI):
                for u in range(EU):
                    r = rows[u] * NEI + k
                    for c in range(NCH):
                        hv = buf[r, pl.ds(c * 16, 16)]
                        qv = buf[r, pl.ds(HIDDEN + c * 16, 16)]
                        acc_s[u][c] = acc_s[u][c] + hv
                        acc_g[u][c] = acc_g[u][c] + hv / (pe[u][c] + qv)
            for u in range(EU):
                for c in range(NCH):
                    obuf[rows[u], pl.ds(c * 16, 16)] = acc_s[u][c]
                    obuf[rows[u], pl.ds(HIDDEN + c * 16, 16)] = (
                        pe[u][c] * acc_g[u][c])

    @pl.loop(0, NCHUNK)
    def _chunk(ch):
        i0 = wid * (EPW * NEI) + ch * (CHE * NEI)
        e0 = wid * EPW + ch * CHE
        pltpu.async_copy(bgf_hbm.at[pl.ds(i0, CHE * NEI)], idx_v, semi)
        pltpu.async_copy(nr1_hbm.at[pl.ds(e0, CHE)], nr1_v, semn)
        pltpu.make_async_copy(bgf_hbm.at[pl.ds(i0, CHE * NEI)], idx_v,
                              semi).wait()
        for j in range(AH):
            pltpu.async_copy(hcat_hbm.at[idx_v.at[pl.ds(j * IBLK, IBLK)]],
                             rows_v.at[j], sems[j])
        pltpu.make_async_copy(nr1_hbm.at[pl.ds(e0, CHE)], nr1_v,
                              semn).wait()

        @pl.loop(0, CHD, step=NB)
        def _block(b):
            for par in range(NB):
                bb = b + par

                @pl.when(bb < CHD)
                def _():
                    pltpu.make_async_copy(
                        hcat_hbm.at[idx_v.at[pl.ds(0, IBLK)]],
                        rows_v.at[par], sems[par]).wait()

                    @pl.when(bb + AH < CHD)
                    def _():
                        pltpu.async_copy(
                            hcat_hbm.at[idx_v.at[pl.ds((bb + AH) * IBLK, IBLK)]],
                            rows_v.at[(par + AH) % NB], sems[(par + AH) % NB])

                    @pl.when(ch * CHD + bb >= NB)
                    def _():
                        pltpu.make_async_copy(
                            outc_v.at[par], sumc_hbm.at[pl.ds(e0, GBD)],
                            semos[par]).wait()

                    _compute(bb, rows_v.at[par], outc_v.at[par])
                    pltpu.async_copy(
                        outc_v.at[par],
                        sumc_hbm.at[pl.ds(e0 + bb * GBD, GBD)], semos[par])

    e0l = wid * EPW
    for par in range(NB):
        pltpu.make_async_copy(outc_v.at[par], sumc_hbm.at[pl.ds(e0l, GBD)],
                              semos[par]).wait()


NP = 10240
NPW = NP // NW
RBLK = NPW // GB


@functools.partial(
    pl.kernel,
    out_type=jax.ShapeDtypeStruct((NP, HIDDEN), F32),
    mesh=_sc_mesh,
    scratch_types=[
        pltpu.VMEM((NPW * NEI,), jnp.int32),
        pltpu.VMEM((2, 128, 2 * HIDDEN), F32),
        pltpu.VMEM((NPW, HIDDEN), F32),
        pltpu.SemaphoreType.DMA,
        pltpu.SemaphoreType.DMA,
    ],
)
def _sc_readout(h_hbm, agf_hbm, nm_hbm, idx_v, rows_v, outn_v, sem0, sem1):
    wid = lax.axis_index("c") * NS + lax.axis_index("s")
    sems = (sem0, sem1)
    pltpu.sync_copy(agf_hbm.at[pl.ds(wid * (NPW * NEI), NPW * NEI)], idx_v)
    pltpu.async_copy(h_hbm.at[idx_v.at[pl.ds(0, 128)]], rows_v.at[0], sem0)

    @pl.loop(0, RBLK, step=2)
    def _block(b):
        for par in range(2):
            bb = b + par
            pltpu.make_async_copy(h_hbm.at[idx_v.at[pl.ds(0, 128)]],
                                  rows_v.at[par], sems[par]).wait()

            @pl.when(bb + 1 < RBLK)
            def _():
                pltpu.async_copy(h_hbm.at[idx_v.at[pl.ds((bb + 1) * 128, 128)]],
                                 rows_v.at[1 - par], sems[1 - par])

            @pl.loop(0, GB)
            def _node(nn):
                orow = bb * GB + nn
                NCH = HIDDEN // 16
                acc = [jnp.zeros((16,), F32) for _ in range(NCH)]
                for k in range(NEI):
                    for c in range(NCH):
                        acc[c] = acc[c] + rows_v[par, nn * NEI + k,
                                                 pl.ds(c * 16, 16)]
                for c in range(NCH):
                    outn_v[orow, pl.ds(c * 16, 16)] = acc[c]

    pltpu.sync_copy(outn_v, nm_hbm.at[pl.ds(wid * NPW, NPW)])


BE = 2000


def _zero_row0(x, pid):
    ri = lax.broadcasted_iota(jnp.int32, x.shape, 0)
    return jnp.where((ri == 0) & (pid == 0), 0.0, x)


def _pre_body(fm_ref, wz1_ref, wr_ref, wh1_ref, ur_ref, bz_ref, bur_ref,
              bh_ref, az_ref, nr1_ref, ah_ref, hcat_ref):
    fm = fm_ref[...]
    az = jnp.dot(fm, wz1_ref[...], preferred_element_type=F32) + bz_ref[...]
    nr1 = jnp.exp(jnp.dot(fm, wr_ref[...], preferred_element_type=F32)
                  + bur_ref[...])
    ah = jnp.dot(fm, wh1_ref[...], preferred_element_type=F32) + bh_ref[...]
    h1 = jax.nn.sigmoid(az) * jnp.tanh(ah)
    h1 = _zero_row0(h1, pl.program_id(0))
    az_ref[...] = az
    nr1_ref[...] = nr1
    ah_ref[...] = ah
    hu = jnp.dot(h1, ur_ref[...], preferred_element_type=F32)
    hcat_ref[...] = jnp.concatenate([h1, jnp.exp(-hu)], axis=1)


def _precompute(fmess, wz1t, wrt, wh1t, urt, bz, bur, bh):
    grid = (E // BE,)
    row = lambda i: (i, 0)
    fix = lambda i: (0, 0)
    return pl.pallas_call(
        _pre_body,
        grid=grid,
        in_specs=[
            pl.BlockSpec((BE, INPUT), row),
            pl.BlockSpec((INPUT, HIDDEN), fix),
            pl.BlockSpec((INPUT, HIDDEN), fix),
            pl.BlockSpec((INPUT, HIDDEN), fix),
            pl.BlockSpec((HIDDEN, HIDDEN), fix),
            pl.BlockSpec((1, HIDDEN), fix),
            pl.BlockSpec((1, HIDDEN), fix),
            pl.BlockSpec((1, HIDDEN), fix),
        ],
        out_specs=[
            pl.BlockSpec((BE, HIDDEN), row),
            pl.BlockSpec((BE, HIDDEN), row),
            pl.BlockSpec((BE, HIDDEN), row),
            pl.BlockSpec((BE, 2 * HIDDEN), row),
        ],
        out_shape=[
            jax.ShapeDtypeStruct((E, HIDDEN), F32),
            jax.ShapeDtypeStruct((E, HIDDEN), F32),
            jax.ShapeDtypeStruct((E, HIDDEN), F32),
            jax.ShapeDtypeStruct((E, 2 * HIDDEN), F32),
        ],
    )(fmess, wz1t, wrt, wh1t, urt, bz, bur, bh)


def _upd_body(final, sc_ref, az_ref, ah_ref, wz2_ref, wh2_ref,
              ur_ref, h_ref, hcat_ref=None):
    sh = sc_ref[:, :HIDDEN]
    sg = sc_ref[:, HIDDEN:]
    z = jax.nn.sigmoid(az_ref[...] + jnp.dot(sh, wz2_ref[...],
                                             preferred_element_type=F32))
    pre = jnp.tanh(ah_ref[...] + jnp.dot(sg, wh2_ref[...],
                                         preferred_element_type=F32))
    h = (1.0 - z) * sh + z * pre
    h = _zero_row0(h, pl.program_id(0))
    h_ref[...] = h
    if not final:
        hu = jnp.dot(h, ur_ref[...], preferred_element_type=F32)
        hcat_ref[...] = jnp.concatenate([h, jnp.exp(-hu)], axis=1)


def _update(final, sumc, az, ah, wz2t, wh2t, urt):
    grid = (E // BE,)
    row = lambda i: (i, 0)
    fix = lambda i: (0, 0)
    out_specs = [pl.BlockSpec((BE, HIDDEN), row)]
    out_shape = [jax.ShapeDtypeStruct((E, HIDDEN), F32)]
    if not final:
        out_specs.append(pl.BlockSpec((BE, 2 * HIDDEN), row))
        out_shape.append(jax.ShapeDtypeStruct((E, 2 * HIDDEN), F32))
    return pl.pallas_call(
        functools.partial(_upd_body, final),
        grid=grid,
        in_specs=[
            pl.BlockSpec((BE, 2 * HIDDEN), row),
            pl.BlockSpec((BE, HIDDEN), row),
            pl.BlockSpec((BE, HIDDEN), row),
            pl.BlockSpec((HIDDEN, HIDDEN), fix),
            pl.BlockSpec((HIDDEN, HIDDEN), fix),
            pl.BlockSpec((HIDDEN, HIDDEN), fix),
        ],
        out_specs=out_specs,
        out_shape=out_shape,
    )(sumc, az, ah, wz2t, wh2t, urt)


BN = 1024


def _ro_body(fn_ref, nm_ref, mask_ref, wo1_ref, wo2_ref, bo_ref, out_ref):
    acc = jnp.dot(fn_ref[...], wo1_ref[...], preferred_element_type=F32)
    acc = acc + jnp.dot(nm_ref[...], wo2_ref[...], preferred_element_type=F32)
    acc = acc + bo_ref[...]
    out_ref[...] = jnp.maximum(acc, 0.0) * mask_ref[...]


def _readout(fnode_p, nm, maskb, wo1t, wo2t, bo):
    grid = (NP // BN,)
    row = lambda i: (i, 0)
    fix = lambda i: (0, 0)
    return pl.pallas_call(
        _ro_body,
        grid=grid,
        in_specs=[
            pl.BlockSpec((BN, NODE_FDIM), row),
            pl.BlockSpec((BN, HIDDEN), row),
            pl.BlockSpec((BN, HIDDEN), row),
            pl.BlockSpec((NODE_FDIM, HIDDEN), fix),
            pl.BlockSpec((HIDDEN, HIDDEN), fix),
            pl.BlockSpec((1, HIDDEN), fix),
        ],
        out_specs=pl.BlockSpec((BN, HIDDEN), row),
        out_shape=jax.ShapeDtypeStruct((NP, HIDDEN), F32),
    )(fnode_p, nm, maskb, wo1t, wo2t, bo)


def kernel(fnode, fmess, agraph, bgraph, mask,
           Wz_w, Wz_b, Wr_w, Ur_w, Ur_b, Wh_w, Wh_b, Wo_w, Wo_b):
    wz1t = Wz_w[:, :INPUT].T
    wz2t = Wz_w[:, INPUT:].T
    wh1t = Wh_w[:, :INPUT].T
    wh2t = Wh_w[:, INPUT:].T
    wrt = Wr_w.T
    urt = Ur_w.T
    wo1t = Wo_w[:, :NODE_FDIM].T
    wo2t = Wo_w[:, NODE_FDIM:].T
    bz = Wz_b.reshape(1, HIDDEN)
    bur = Ur_b.reshape(1, HIDDEN)
    bh = Wh_b.reshape(1, HIDDEN)
    bo = Wo_b.reshape(1, HIDDEN)

    bgf = bgraph.reshape(E * NEI)
    agraph_p = jnp.pad(agraph, ((0, NP - N), (0, 0)))
    agf = agraph_p.reshape(NP * NEI)
    fnode_p = jnp.pad(fnode, ((0, NP - N), (0, 0)))
    maskb = jnp.pad(jnp.broadcast_to(mask, (N, HIDDEN)), ((0, NP - N), (0, 0)))

    az, nr1, ah, hcat = _precompute(fmess, wz1t, wrt, wh1t, urt, bz, bur, bh)

    sumc = _sc_depth(hcat, bgf, nr1)
    h, hcat = _update(False, sumc, az, ah, wz2t, wh2t, urt)

    sumc = _sc_depth(hcat, bgf, nr1)
    h, hcat = _update(False, sumc, az, ah, wz2t, wh2t, urt)

    nm = _sc_readout(hcat, agf)
    node_hiddens = _readout(fnode_p, nm, maskb, wo1t, wo2t, bo)[:N]
    return (node_hiddens, h)


# speedup vs baseline: 1.0186x; 1.0186x over previous
import functools

import jax
import jax.numpy as jnp
from jax import lax
from jax.experimental import pallas as pl
from jax.experimental.pallas import tpu as pltpu
from jax.experimental.pallas import tpu_sc as plsc

N = 10000
E = 160000
NEI = 16
NODE_FDIM = 128
INPUT = 128
HIDDEN = 64

F32 = jnp.float32

NC = 2
NS = 16
NW = NC * NS

EPW = E // NW
GB = 8
CHE = 200
NCHUNK = EPW // CHE
GBD = 8
IBLK = GBD * NEI
CHD = CHE // GBD
NB = 4
AH = 3

_sc_mesh = plsc.VectorSubcoreMesh(core_axis_name="c", subcore_axis_name="s")


@functools.partial(
    pl.kernel,
    out_type=jax.ShapeDtypeStruct((E, 2 * HIDDEN), F32),
    mesh=_sc_mesh,
    scratch_types=[
        pltpu.VMEM((CHE * NEI,), jnp.int32),
        pltpu.VMEM((CHE, HIDDEN), F32),
        pltpu.VMEM((NB, IBLK, 2 * HIDDEN), F32),
        pltpu.VMEM((NB, GBD, 2 * HIDDEN), F32),
        pltpu.SemaphoreType.DMA,
        pltpu.SemaphoreType.DMA,
        pltpu.SemaphoreType.DMA,
        pltpu.SemaphoreType.DMA,
        pltpu.SemaphoreType.DMA,
        pltpu.SemaphoreType.DMA,
        pltpu.SemaphoreType.DMA,
        pltpu.SemaphoreType.DMA,
        pltpu.SemaphoreType.DMA,
        pltpu.SemaphoreType.DMA,
    ],
)
def _sc_depth(hcat_hbm, bgf_hbm, nr1_hbm, sumc_hbm,
              idx_v, nr1_v, rows_v, outc_v,
              sem0, sem1, sem2, sem3, semo0, semo1, semo2, semo3,
              semi, semn):
    wid = lax.axis_index("c") * NS + lax.axis_index("s")
    sems = (sem0, sem1, sem2, sem3)
    semos = (semo0, semo1, semo2, semo3)

    def _compute(b, buf, obuf):
        NCH = HIDDEN // 16
        EU = 2

        @pl.loop(0, GBD, step=EU)
        def _edge(e):
            rows = [e + u for u in range(EU)]
            pe = [[nr1_v[b * GBD + ee, pl.ds(c * 16, 16)] for c in range(NCH)]
                  for ee in rows]
            acc_s = [[jnp.zeros((16,), F32) for _ in range(NCH)]
                     for _ in range(EU)]
            acc_g = [[jnp.zeros((16,), F32) for _ in range(NCH)]
                     for _ in range(EU)]
            for k in range(NEI):
                for u in range(EU):
                    r = rows[u] * NEI + k
                    for c in range(NCH):
                        hv = buf[r, pl.ds(c * 16, 16)]
                        qv = buf[r, pl.ds(HIDDEN + c * 16, 16)]
                        acc_s[u][c] = acc_s[u][c] + hv
                        acc_g[u][c] = acc_g[u][c] + hv / (pe[u][c] + qv)
            for u in range(EU):
                for c in range(NCH):
                    obuf[rows[u], pl.ds(c * 16, 16)] = acc_s[u][c]
                    obuf[rows[u], pl.ds(HIDDEN + c * 16, 16)] = (
                        pe[u][c] * acc_g[u][c])

    @pl.loop(0, NCHUNK)
    def _chunk(ch):
        i0 = wid * (EPW * NEI) + ch * (CHE * NEI)
        e0 = wid * EPW + ch * CHE
        pltpu.async_copy(bgf_hbm.at[pl.ds(i0, CHE * NEI)], idx_v, semi)
        pltpu.async_copy(nr1_hbm.at[pl.ds(e0, CHE)], nr1_v, semn)
        pltpu.make_async_copy(bgf_hbm.at[pl.ds(i0, CHE * NEI)], idx_v,
                              semi).wait()
        for j in range(AH):
            pltpu.async_copy(hcat_hbm.at[idx_v.at[pl.ds(j * IBLK, IBLK)]],
                             rows_v.at[j], sems[j])
        pltpu.make_async_copy(nr1_hbm.at[pl.ds(e0, CHE)], nr1_v,
                              semn).wait()

        @pl.loop(0, CHD, step=NB)
        def _block(b):
            for par in range(NB):
                bb = b + par

                @pl.when(bb < CHD)
                def _():
                    pltpu.make_async_copy(
                        hcat_hbm.at[idx_v.at[pl.ds(0, IBLK)]],
                        rows_v.at[par], sems[par]).wait()

                    @pl.when(bb + AH < CHD)
                    def _():
                        pltpu.async_copy(
                            hcat_hbm.at[idx_v.at[pl.ds((bb + AH) * IBLK, IBLK)]],
                            rows_v.at[(par + AH) % NB], sems[(par + AH) % NB])

                    @pl.when(ch * CHD + bb >= NB)
                    def _():
                        pltpu.make_async_copy(
                            outc_v.at[par], sumc_hbm.at[pl.ds(e0, GBD)],
                            semos[par]).wait()

                    _compute(bb, rows_v.at[par], outc_v.at[par])
                    pltpu.async_copy(
                        outc_v.at[par],
                        sumc_hbm.at[pl.ds(e0 + bb * GBD, GBD)], semos[par])

    e0l = wid * EPW
    for par in range(NB):
        pltpu.make_async_copy(outc_v.at[par], sumc_hbm.at[pl.ds(e0l, GBD)],
                              semos[par]).wait()


NP = 10240
NPW = NP // NW
RBLK = NPW // GB


@functools.partial(
    pl.kernel,
    out_type=jax.ShapeDtypeStruct((NP, HIDDEN), F32),
    mesh=_sc_mesh,
    scratch_types=[
        pltpu.VMEM((NPW * NEI,), jnp.int32),
        pltpu.VMEM((2, 128, 2 * HIDDEN), F32),
        pltpu.VMEM((NPW, HIDDEN), F32),
        pltpu.SemaphoreType.DMA,
        pltpu.SemaphoreType.DMA,
    ],
)
def _sc_readout(h_hbm, agf_hbm, nm_hbm, idx_v, rows_v, outn_v, sem0, sem1):
    wid = lax.axis_index("c") * NS + lax.axis_index("s")
    sems = (sem0, sem1)
    pltpu.sync_copy(agf_hbm.at[pl.ds(wid * (NPW * NEI), NPW * NEI)], idx_v)
    pltpu.async_copy(h_hbm.at[idx_v.at[pl.ds(0, 128)]], rows_v.at[0], sem0)

    @pl.loop(0, RBLK, step=2)
    def _block(b):
        for par in range(2):
            bb = b + par
            pltpu.make_async_copy(h_hbm.at[idx_v.at[pl.ds(0, 128)]],
                                  rows_v.at[par], sems[par]).wait()

            @pl.when(bb + 1 < RBLK)
            def _():
                pltpu.async_copy(h_hbm.at[idx_v.at[pl.ds((bb + 1) * 128, 128)]],
                                 rows_v.at[1 - par], sems[1 - par])

            @pl.loop(0, GB)
            def _node(nn):
                orow = bb * GB + nn
                NCH = HIDDEN // 16
                acc = [jnp.zeros((16,), F32) for _ in range(NCH)]
                for k in range(NEI):
                    for c in range(NCH):
                        acc[c] = acc[c] + rows_v[par, nn * NEI + k,
                                                 pl.ds(c * 16, 16)]
                for c in range(NCH):
                    outn_v[orow, pl.ds(c * 16, 16)] = acc[c]

    pltpu.sync_copy(outn_v, nm_hbm.at[pl.ds(wid * NPW, NPW)])


BE = 2000


def _zero_row0(x, pid):
    ri = lax.broadcasted_iota(jnp.int32, x.shape, 0)
    return jnp.where((ri == 0) & (pid == 0), 0.0, x)


def _pre_body(fm_ref, wz1_ref, wr_ref, wh1_ref, ur_ref, bz_ref, bur_ref,
              bh_ref, az_ref, nr1_ref, ah_ref, hcat_ref):
    fm = fm_ref[...]
    az = jnp.dot(fm, wz1_ref[...], preferred_element_type=F32) + bz_ref[...]
    nr1 = jnp.exp(jnp.dot(fm, wr_ref[...], preferred_element_type=F32)
                  + bur_ref[...])
    ah = jnp.dot(fm, wh1_ref[...], preferred_element_type=F32) + bh_ref[...]
    h1 = jax.nn.sigmoid(az) * jnp.tanh(ah)
    h1 = _zero_row0(h1, pl.program_id(0))
    az_ref[...] = az
    nr1_ref[...] = nr1
    ah_ref[...] = ah
    hu = jnp.dot(h1, ur_ref[...], preferred_element_type=F32)
    hcat_ref[...] = jnp.concatenate([h1, jnp.exp(-hu)], axis=1)


def _precompute(fmess, wz1t, wrt, wh1t, urt, bz, bur, bh):
    grid = (E // BE,)
    row = lambda i: (i, 0)
    fix = lambda i: (0, 0)
    return pl.pallas_call(
        _pre_body,
        grid=grid,
        in_specs=[
            pl.BlockSpec((BE, INPUT), row),
            pl.BlockSpec((INPUT, HIDDEN), fix),
            pl.BlockSpec((INPUT, HIDDEN), fix),
            pl.BlockSpec((INPUT, HIDDEN), fix),
            pl.BlockSpec((HIDDEN, HIDDEN), fix),
            pl.BlockSpec((1, HIDDEN), fix),
            pl.BlockSpec((1, HIDDEN), fix),
            pl.BlockSpec((1, HIDDEN), fix),
        ],
        out_specs=[
            pl.BlockSpec((BE, HIDDEN), row),
            pl.BlockSpec((BE, HIDDEN), row),
            pl.BlockSpec((BE, HIDDEN), row),
            pl.BlockSpec((BE, 2 * HIDDEN), row),
        ],
        out_shape=[
            jax.ShapeDtypeStruct((E, HIDDEN), F32),
            jax.ShapeDtypeStruct((E, HIDDEN), F32),
            jax.ShapeDtypeStruct((E, HIDDEN), F32),
            jax.ShapeDtypeStruct((E, 2 * HIDDEN), F32),
        ],
    )(fmess, wz1t, wrt, wh1t, urt, bz, bur, bh)


def _upd_body(final, sc_ref, az_ref, ah_ref, wz2_ref, wh2_ref,
              ur_ref, h_ref, hcat_ref=None):
    sh = sc_ref[:, :HIDDEN]
    sg = sc_ref[:, HIDDEN:]
    z = jax.nn.sigmoid(az_ref[...] + jnp.dot(sh, wz2_ref[...],
                                             preferred_element_type=F32))
    pre = jnp.tanh(ah_ref[...] + jnp.dot(sg, wh2_ref[...],
                                         preferred_element_type=F32))
    h = (1.0 - z) * sh + z * pre
    h = _zero_row0(h, pl.program_id(0))
    h_ref[...] = h
    if not final:
        hu = jnp.dot(h, ur_ref[...], preferred_element_type=F32)
        hcat_ref[...] = jnp.concatenate([h, jnp.exp(-hu)], axis=1)


def _update(final, sumc, az, ah, wz2t, wh2t, urt):
    grid = (E // BE,)
    row = lambda i: (i, 0)
    fix = lambda i: (0, 0)
    out_specs = [pl.BlockSpec((BE, HIDDEN), row)]
    out_shape = [jax.ShapeDtypeStruct((E, HIDDEN), F32)]
    if not final:
        out_specs.append(pl.BlockSpec((BE, 2 * HIDDEN), row))
        out_shape.append(jax.ShapeDtypeStruct((E, 2 * HIDDEN), F32))
    return pl.pallas_call(
        functools.partial(_upd_body, final),
        grid=grid,
        in_specs=[
            pl.BlockSpec((BE, 2 * HIDDEN), row),
            pl.BlockSpec((BE, HIDDEN), row),
            pl.BlockSpec((BE, HIDDEN), row),
            pl.BlockSpec((HIDDEN, HIDDEN), fix),
            pl.BlockSpec((HIDDEN, HIDDEN), fix),
            pl.BlockSpec((HIDDEN, HIDDEN), fix),
        ],
        out_specs=out_specs,
        out_shape=out_shape,
    )(sumc, az, ah, wz2t, wh2t, urt)


BN = 1024


def _ro_body(fn_ref, nm_ref, mask_ref, wo1_ref, wo2_ref, bo_ref, out_ref):
    acc = jnp.dot(fn_ref[...], wo1_ref[...], preferred_element_type=F32)
    acc = acc + jnp.dot(nm_ref[...], wo2_ref[...], preferred_element_type=F32)
    acc = acc + bo_ref[...]
    out_ref[...] = jnp.maximum(acc, 0.0) * mask_ref[...]


def _readout(fnode_p, nm, maskb, wo1t, wo2t, bo):
    grid = (NP // BN,)
    row = lambda i: (i, 0)
    fix = lambda i: (0, 0)
    return pl.pallas_call(
        _ro_body,
        grid=grid,
        in_specs=[
            pl.BlockSpec((BN, NODE_FDIM), row),
            pl.BlockSpec((BN, HIDDEN), row),
            pl.BlockSpec((BN, HIDDEN), row),
            pl.BlockSpec((NODE_FDIM, HIDDEN), fix),
            pl.BlockSpec((HIDDEN, HIDDEN), fix),
            pl.BlockSpec((1, HIDDEN), fix),
        ],
        out_specs=pl.BlockSpec((BN, HIDDEN), row),
        out_shape=jax.ShapeDtypeStruct((NP, HIDDEN), F32),
    )(fnode_p, nm, maskb, wo1t, wo2t, bo)


def kernel(fnode, fmess, agraph, bgraph, mask,
           Wz_w, Wz_b, Wr_w, Ur_w, Ur_b, Wh_w, Wh_b, Wo_w, Wo_b):
    wz1t = Wz_w[:, :INPUT].T
    wz2t = Wz_w[:, INPUT:].T
    wh1t = Wh_w[:, :INPUT].T
    wh2t = Wh_w[:, INPUT:].T
    wrt = Wr_w.T
    urt = Ur_w.T
    wo1t = Wo_w[:, :NODE_FDIM].T
    wo2t = Wo_w[:, NODE_FDIM:].T
    bz = Wz_b.reshape(1, HIDDEN)
    bur = Ur_b.reshape(1, HIDDEN)
    bh = Wh_b.reshape(1, HIDDEN)
    bo = Wo_b.reshape(1, HIDDEN)

    bgf = bgraph.reshape(E * NEI)
    agraph_p = jnp.pad(agraph, ((0, NP - N), (0, 0)))
    agf = agraph_p.reshape(NP * NEI)
    fnode_p = jnp.pad(fnode, ((0, NP - N), (0, 0)))
    maskb = jnp.pad(jnp.broadcast_to(mask, (N, HIDDEN)), ((0, NP - N), (0, 0)))

    az, nr1, ah, hcat = _precompute(fmess, wz1t, wrt, wh1t, urt, bz, bur, bh)

    sumc = _sc_depth(hcat, bgf, nr1)
    h, hcat = _update(False, sumc, az, ah, wz2t, wh2t, urt)

    sumc = _sc_depth(hcat, bgf, nr1)
    h, hcat = _update(False, sumc, az, ah, wz2t, wh2t, urt)

    nm = _sc_readout(hcat, agf)
    node_hiddens = _readout(fnode_p, nm, maskb, wo1t, wo2t, bo)[:N]
    return (node_hiddens, h)


# speedup vs baseline: 1.0314x; 1.0126x over previous
import functools

import jax
import jax.numpy as jnp
from jax import lax
from jax.experimental import pallas as pl
from jax.experimental.pallas import tpu as pltpu
from jax.experimental.pallas import tpu_sc as plsc

N = 10000
E = 160000
NEI = 16
NODE_FDIM = 128
INPUT = 128
HIDDEN = 64

F32 = jnp.float32

NC = 2
NS = 16
NW = NC * NS

EPW = E // NW
GB = 8
CHE = 200
NCHUNK = EPW // CHE
GBD = 8
IBLK = GBD * NEI
CHD = CHE // GBD
NB = 4
AH = 3

_sc_mesh = plsc.VectorSubcoreMesh(core_axis_name="c", subcore_axis_name="s")


@functools.partial(
    pl.kernel,
    out_type=jax.ShapeDtypeStruct((E, 2 * HIDDEN), F32),
    mesh=_sc_mesh,
    scratch_types=[
        pltpu.VMEM((CHE * NEI,), jnp.int32),
        pltpu.VMEM((CHE, HIDDEN), F32),
        pltpu.VMEM((NB, IBLK, 2 * HIDDEN), F32),
        pltpu.VMEM((NB, GBD, 2 * HIDDEN), F32),
        pltpu.SemaphoreType.DMA,
        pltpu.SemaphoreType.DMA,
        pltpu.SemaphoreType.DMA,
        pltpu.SemaphoreType.DMA,
        pltpu.SemaphoreType.DMA,
        pltpu.SemaphoreType.DMA,
        pltpu.SemaphoreType.DMA,
        pltpu.SemaphoreType.DMA,
        pltpu.SemaphoreType.DMA,
        pltpu.SemaphoreType.DMA,
    ],
)
def _sc_depth(hcat_hbm, bgf_hbm, nr1_hbm, sumc_hbm,
              idx_v, nr1_v, rows_v, outc_v,
              sem0, sem1, sem2, sem3, semo0, semo1, semo2, semo3,
              semi, semn):
    wid = lax.axis_index("c") * NS + lax.axis_index("s")
    sems = (sem0, sem1, sem2, sem3)
    semos = (semo0, semo1, semo2, semo3)

    def _compute(b, buf, obuf):
        NCH = HIDDEN // 16
        EU = 2

        @pl.loop(0, GBD, step=EU)
        def _edge(e):
            rows = [e + u for u in range(EU)]
            pe = [[nr1_v[b * GBD + ee, pl.ds(c * 16, 16)] for c in range(NCH)]
                  for ee in rows]
            acc_s = [[jnp.zeros((16,), F32) for _ in range(NCH)]
                     for _ in range(EU)]
            acc_g = [[jnp.zeros((16,), F32) for _ in range(NCH)]
                     for _ in range(EU)]
            for k in range(NEI):
                for u in range(EU):
                    r = rows[u] * NEI + k
                    for c in range(NCH):
                        hv = buf[r, pl.ds(c * 16, 16)]
                        qv = buf[r, pl.ds(HIDDEN + c * 16, 16)]
                        acc_s[u][c] = acc_s[u][c] + hv
                        acc_g[u][c] = acc_g[u][c] + hv / (pe[u][c] + qv)
            for u in range(EU):
                for c in range(NCH):
                    obuf[rows[u], pl.ds(c * 16, 16)] = acc_s[u][c]
                    obuf[rows[u], pl.ds(HIDDEN + c * 16, 16)] = (
                        pe[u][c] * acc_g[u][c])

    @pl.loop(0, NCHUNK)
    def _chunk(ch):
        i0 = wid * (EPW * NEI) + ch * (CHE * NEI)
        e0 = wid * EPW + ch * CHE
        pltpu.async_copy(bgf_hbm.at[pl.ds(i0, CHE * NEI)], idx_v, semi)
        pltpu.async_copy(nr1_hbm.at[pl.ds(e0, CHE)], nr1_v, semn)
        pltpu.make_async_copy(bgf_hbm.at[pl.ds(i0, CHE * NEI)], idx_v,
                              semi).wait()
        for j in range(AH):
            pltpu.async_copy(hcat_hbm.at[idx_v.at[pl.ds(j * IBLK, IBLK)]],
                             rows_v.at[j], sems[j])
        pltpu.make_async_copy(nr1_hbm.at[pl.ds(e0, CHE)], nr1_v,
                              semn).wait()

        @pl.loop(0, CHD, step=NB)
        def _block(b):
            for par in range(NB):
                bb = b + par

                @pl.when(bb < CHD)
                def _():
                    pltpu.make_async_copy(
                        hcat_hbm.at[idx_v.at[pl.ds(0, IBLK)]],
                        rows_v.at[par], sems[par]).wait()

                    @pl.when(bb + AH < CHD)
                    def _():
                        pltpu.async_copy(
                            hcat_hbm.at[idx_v.at[pl.ds((bb + AH) * IBLK, IBLK)]],
                            rows_v.at[(par + AH) % NB], sems[(par + AH) % NB])

                    @pl.when(ch * CHD + bb >= NB)
                    def _():
                        pltpu.make_async_copy(
                            outc_v.at[par], sumc_hbm.at[pl.ds(e0, GBD)],
                            semos[par]).wait()

                    _compute(bb, rows_v.at[par], outc_v.at[par])
                    pltpu.async_copy(
                        outc_v.at[par],
                        sumc_hbm.at[pl.ds(e0 + bb * GBD, GBD)], semos[par])

    e0l = wid * EPW
    for par in range(NB):
        pltpu.make_async_copy(outc_v.at[par], sumc_hbm.at[pl.ds(e0l, GBD)],
                              semos[par]).wait()


NP = 10240
NPW = NP // NW
RBLK = NPW // GB


@functools.partial(
    pl.kernel,
    out_type=jax.ShapeDtypeStruct((NP, HIDDEN), F32),
    mesh=_sc_mesh,
    scratch_types=[
        pltpu.VMEM((NPW * NEI,), jnp.int32),
        pltpu.VMEM((4, 128, 2 * HIDDEN), F32),
        pltpu.VMEM((4, GB, HIDDEN), F32),
        pltpu.SemaphoreType.DMA,
        pltpu.SemaphoreType.DMA,
        pltpu.SemaphoreType.DMA,
        pltpu.SemaphoreType.DMA,
        pltpu.SemaphoreType.DMA,
        pltpu.SemaphoreType.DMA,
        pltpu.SemaphoreType.DMA,
        pltpu.SemaphoreType.DMA,
    ],
)
def _sc_readout(h_hbm, agf_hbm, nm_hbm, idx_v, rows_v, outn_v,
                sem0, sem1, sem2, sem3, semo0, semo1, semo2, semo3):
    wid = lax.axis_index("c") * NS + lax.axis_index("s")
    sems = (sem0, sem1, sem2, sem3)
    semos = (semo0, semo1, semo2, semo3)
    n0 = wid * NPW
    pltpu.sync_copy(agf_hbm.at[pl.ds(n0 * NEI, NPW * NEI)], idx_v)
    for j in range(3):
        pltpu.async_copy(h_hbm.at[idx_v.at[pl.ds(j * 128, 128)]],
                         rows_v.at[j], sems[j])

    @pl.loop(0, RBLK, step=4)
    def _block(b):
        for par in range(4):
            bb = b + par
            pltpu.make_async_copy(h_hbm.at[idx_v.at[pl.ds(0, 128)]],
                                  rows_v.at[par], sems[par]).wait()

            @pl.when(bb + 3 < RBLK)
            def _():
                pltpu.async_copy(h_hbm.at[idx_v.at[pl.ds((bb + 3) * 128, 128)]],
                                 rows_v.at[(par + 3) % 4], sems[(par + 3) % 4])

            @pl.when(bb >= 4)
            def _():
                pltpu.make_async_copy(outn_v.at[par],
                                      nm_hbm.at[pl.ds(n0, GB)],
                                      semos[par]).wait()

            @pl.loop(0, GB)
            def _node(nn):
                NCH = HIDDEN // 16
                acc = [jnp.zeros((16,), F32) for _ in range(NCH)]
                for k in range(NEI):
                    for c in range(NCH):
                        acc[c] = acc[c] + rows_v[par, nn * NEI + k,
                                                 pl.ds(c * 16, 16)]
                for c in range(NCH):
                    outn_v[par, nn, pl.ds(c * 16, 16)] = acc[c]

            pltpu.async_copy(outn_v.at[par],
                             nm_hbm.at[pl.ds(n0 + bb * GB, GB)], semos[par])

    for par in range(4):
        pltpu.make_async_copy(outn_v.at[par], nm_hbm.at[pl.ds(n0, GB)],
                              semos[par]).wait()


BE = 2000


def _zero_row0(x, pid):
    ri = lax.broadcasted_iota(jnp.int32, x.shape, 0)
    return jnp.where((ri == 0) & (pid == 0), 0.0, x)


def _pre_body(fm_ref, wz1_ref, wr_ref, wh1_ref, ur_ref, bz_ref, bur_ref,
              bh_ref, az_ref, nr1_ref, ah_ref, hcat_ref):
    fm = fm_ref[...]
    az = jnp.dot(fm, wz1_ref[...], preferred_element_type=F32) + bz_ref[...]
    nr1 = jnp.exp(jnp.dot(fm, wr_ref[...], preferred_element_type=F32)
                  + bur_ref[...])
    ah = jnp.dot(fm, wh1_ref[...], preferred_element_type=F32) + bh_ref[...]
    h1 = jax.nn.sigmoid(az) * jnp.tanh(ah)
    h1 = _zero_row0(h1, pl.program_id(0))
    az_ref[...] = az
    nr1_ref[...] = nr1
    ah_ref[...] = ah
    hu = jnp.dot(h1, ur_ref[...], preferred_element_type=F32)
    hcat_ref[...] = jnp.concatenate([h1, jnp.exp(-hu)], axis=1)


def _precompute(fmess, wz1t, wrt, wh1t, urt, bz, bur, bh):
    grid = (E // BE,)
    row = lambda i: (i, 0)
    fix = lambda i: (0, 0)
    return pl.pallas_call(
        _pre_body,
        grid=grid,
        in_specs=[
            pl.BlockSpec((BE, INPUT), row),
            pl.BlockSpec((INPUT, HIDDEN), fix),
            pl.BlockSpec((INPUT, HIDDEN), fix),
            pl.BlockSpec((INPUT, HIDDEN), fix),
            pl.BlockSpec((HIDDEN, HIDDEN), fix),
            pl.BlockSpec((1, HIDDEN), fix),
            pl.BlockSpec((1, HIDDEN), fix),
            pl.BlockSpec((1, HIDDEN), fix),
        ],
        out_specs=[
            pl.BlockSpec((BE, HIDDEN), row),
            pl.BlockSpec((BE, HIDDEN), row),
            pl.BlockSpec((BE, HIDDEN), row),
            pl.BlockSpec((BE, 2 * HIDDEN), row),
        ],
        out_shape=[
            jax.ShapeDtypeStruct((E, HIDDEN), F32),
            jax.ShapeDtypeStruct((E, HIDDEN), F32),
            jax.ShapeDtypeStruct((E, HIDDEN), F32),
            jax.ShapeDtypeStruct((E, 2 * HIDDEN), F32),
        ],
    )(fmess, wz1t, wrt, wh1t, urt, bz, bur, bh)


def _upd_body(final, sc_ref, az_ref, ah_ref, wz2_ref, wh2_ref,
              ur_ref, h_ref, hcat_ref=None):
    sh = sc_ref[:, :HIDDEN]
    sg = sc_ref[:, HIDDEN:]
    z = jax.nn.sigmoid(az_ref[...] + jnp.dot(sh, wz2_ref[...],
                                             preferred_element_type=F32))
    pre = jnp.tanh(ah_ref[...] + jnp.dot(sg, wh2_ref[...],
                                         preferred_element_type=F32))
    h = (1.0 - z) * sh + z * pre
    h = _zero_row0(h, pl.program_id(0))
    h_ref[...] = h
    if not final:
        hu = jnp.dot(h, ur_ref[...], preferred_element_type=F32)
        hcat_ref[...] = jnp.concatenate([h, jnp.exp(-hu)], axis=1)


def _update(final, sumc, az, ah, wz2t, wh2t, urt):
    grid = (E // BE,)
    row = lambda i: (i, 0)
    fix = lambda i: (0, 0)
    out_specs = [pl.BlockSpec((BE, HIDDEN), row)]
    out_shape = [jax.ShapeDtypeStruct((E, HIDDEN), F32)]
    if not final:
        out_specs.append(pl.BlockSpec((BE, 2 * HIDDEN), row))
        out_shape.append(jax.ShapeDtypeStruct((E, 2 * HIDDEN), F32))
    return pl.pallas_call(
        functools.partial(_upd_body, final),
        grid=grid,
        in_specs=[
            pl.BlockSpec((BE, 2 * HIDDEN), row),
            pl.BlockSpec((BE, HIDDEN), row),
            pl.BlockSpec((BE, HIDDEN), row),
            pl.BlockSpec((HIDDEN, HIDDEN), fix),
            pl.BlockSpec((HIDDEN, HIDDEN), fix),
            pl.BlockSpec((HIDDEN, HIDDEN), fix),
        ],
        out_specs=out_specs,
        out_shape=out_shape,
    )(sumc, az, ah, wz2t, wh2t, urt)


BN = 1024


def _ro_body(fn_ref, nm_ref, mask_ref, wo1_ref, wo2_ref, bo_ref, out_ref):
    acc = jnp.dot(fn_ref[...], wo1_ref[...], preferred_element_type=F32)
    acc = acc + jnp.dot(nm_ref[...], wo2_ref[...], preferred_element_type=F32)
    acc = acc + bo_ref[...]
    out_ref[...] = jnp.maximum(acc, 0.0) * mask_ref[...]


def _readout(fnode_p, nm, maskb, wo1t, wo2t, bo):
    grid = (NP // BN,)
    row = lambda i: (i, 0)
    fix = lambda i: (0, 0)
    return pl.pallas_call(
        _ro_body,
        grid=grid,
        in_specs=[
            pl.BlockSpec((BN, NODE_FDIM), row),
            pl.BlockSpec((BN, HIDDEN), row),
            pl.BlockSpec((BN, HIDDEN), row),
            pl.BlockSpec((NODE_FDIM, HIDDEN), fix),
            pl.BlockSpec((HIDDEN, HIDDEN), fix),
            pl.BlockSpec((1, HIDDEN), fix),
        ],
        out_specs=pl.BlockSpec((BN, HIDDEN), row),
        out_shape=jax.ShapeDtypeStruct((NP, HIDDEN), F32),
    )(fnode_p, nm, maskb, wo1t, wo2t, bo)


def kernel(fnode, fmess, agraph, bgraph, mask,
           Wz_w, Wz_b, Wr_w, Ur_w, Ur_b, Wh_w, Wh_b, Wo_w, Wo_b):
    wz1t = Wz_w[:, :INPUT].T
    wz2t = Wz_w[:, INPUT:].T
    wh1t = Wh_w[:, :INPUT].T
    wh2t = Wh_w[:, INPUT:].T
    wrt = Wr_w.T
    urt = Ur_w.T
    wo1t = Wo_w[:, :NODE_FDIM].T
    wo2t = Wo_w[:, NODE_FDIM:].T
    bz = Wz_b.reshape(1, HIDDEN)
    bur = Ur_b.reshape(1, HIDDEN)
    bh = Wh_b.reshape(1, HIDDEN)
    bo = Wo_b.reshape(1, HIDDEN)

    bgf = bgraph.reshape(E * NEI)
    agraph_p = jnp.pad(agraph, ((0, NP - N), (0, 0)))
    agf = agraph_p.reshape(NP * NEI)
    fnode_p = jnp.pad(fnode, ((0, NP - N), (0, 0)))
    maskb = jnp.pad(jnp.broadcast_to(mask, (N, HIDDEN)), ((0, NP - N), (0, 0)))

    az, nr1, ah, hcat = _precompute(fmess, wz1t, wrt, wh1t, urt, bz, bur, bh)

    sumc = _sc_depth(hcat, bgf, nr1)
    h, hcat = _update(False, sumc, az, ah, wz2t, wh2t, urt)

    sumc = _sc_depth(hcat, bgf, nr1)
    h, hcat = _update(False, sumc, az, ah, wz2t, wh2t, urt)

    nm = _sc_readout(hcat, agf)
    node_hiddens = _readout(fnode_p, nm, maskb, wo1t, wo2t, bo)[:N]
    return (node_hiddens, h)
